# baseline (speedup 1.0000x reference)
import functools

import jax
import jax.numpy as jnp
from jax import lax
from jax.experimental import pallas as pl
from jax.experimental.pallas import tpu as pltpu
from jax.experimental.pallas import tpu_sc as plsc

BATCH = 4096
SEQ_LEN = 200
D_MODEL = 64

NC = 2
NS = 16
NW = NC * NS

B = BATCH * SEQ_LEN
CHUNK = 128
NB = BATCH // CHUNK
UNITS = SEQ_LEN * NB
U_PER_W = UNITS // NW
B_PER_W = U_PER_W * CHUNK
NBUF = 2

_mesh = plsc.VectorSubcoreMesh(core_axis_name="c", subcore_axis_name="s")


@functools.partial(
    pl.kernel,
    mesh=_mesh,
    compiler_params=pltpu.CompilerParams(
        use_tc_tiling_on_sc=False, needs_layout_passes=False
    ),
    out_type=jax.ShapeDtypeStruct((SEQ_LEN, 8, NB, 8, CHUNK), jnp.float32),
    scratch_types=[
        pltpu.VMEM((B_PER_W,), jnp.int32),
        pltpu.VMEM((NBUF * CHUNK, D_MODEL), jnp.float32),
        pltpu.VMEM((NBUF * 8, 8, CHUNK), jnp.float32),
        pltpu.SemaphoreType.DMA((NBUF,)),
        pltpu.SemaphoreType.DMA((NBUF,)),
    ],
)
def _emb_lookup(idx_hbm, table_hbm, out_hbm, idx_v, rows, tbs, gsem, wsem):
    wid = lax.axis_index("s") * NC + lax.axis_index("c")
    base = wid * B_PER_W

    pltpu.sync_copy(idx_hbm.at[pl.ds(base, B_PER_W)], idx_v)

    iota = lax.iota(jnp.int32, 16)

    def fire(j, b):
        pltpu.async_copy(
            table_hbm.at[idx_v.at[pl.ds(j * CHUNK, CHUNK)]],
            rows.at[pl.ds(b * CHUNK, CHUNK)],
            gsem.at[b],
        )

    def transpose(b):
        for d in range(D_MODEL):
            col = jnp.full((16,), d, jnp.int32)
            for g in range(8):
                v = plsc.load_gather(rows, [iota + (b * CHUNK + 16 * g), col])
                tbs[b * 8 + d // 8, d % 8, pl.ds(16 * g, 16)] = v

    def step(j, b):
        pltpu.make_async_copy(
            table_hbm.at[pl.ds(0, CHUNK)],
            rows.at[pl.ds(b * CHUNK, CHUNK)],
            gsem.at[b],
        ).wait()

        @pl.when(j >= NBUF)
        def _():
            pltpu.make_async_copy(
                table_hbm.at[pl.ds(0, CHUNK)],
                tbs.at[pl.ds(b * 8, 8)],
                wsem.at[b],
            ).wait()

        transpose(b)

        u = wid * U_PER_W + j
        s = u // NB
        c = lax.rem(u, NB)
        pltpu.async_copy(
            tbs.at[pl.ds(b * 8, 8)], out_hbm.at[s, :, c], wsem.at[b]
        )

        @pl.when(j < U_PER_W - NBUF)
        def _():
            fire(j + NBUF, b)

    for b in range(NBUF):
        fire(b, b)

    def group(g, carry):
        for b in range(NBUF):
            step(g * NBUF + b, b)
        return carry

    lax.fori_loop(0, U_PER_W // NBUF, group, 0, unroll=False)

    for b in range(NBUF):
        pltpu.make_async_copy(
            table_hbm.at[pl.ds(0, CHUNK)], tbs.at[pl.ds(b * 8, 8)], wsem.at[b]
        ).wait()


def kernel(char_tokens, table):
    idx = char_tokens.T.reshape(B).astype(jnp.int32)
    out5 = _emb_lookup(idx, table)
    return out5.transpose(2, 4, 0, 1, 3).reshape(BATCH, SEQ_LEN, D_MODEL)


# speedup vs baseline: 1.0014x; 1.0014x over previous
import functools

import jax
import jax.numpy as jnp
from jax import lax
from jax.experimental import pallas as pl
from jax.experimental.pallas import tpu as pltpu
from jax.experimental.pallas import tpu_sc as plsc

BATCH = 4096
SEQ_LEN = 200
D_MODEL = 64

NC = 2
NS = 16
NW = NC * NS

B = BATCH * SEQ_LEN
CHUNK = 128
NB = BATCH // CHUNK
UNITS = SEQ_LEN * NB
U_PER_W = UNITS // NW
B_PER_W = U_PER_W * CHUNK
NBUF = 2

_mesh = plsc.VectorSubcoreMesh(core_axis_name="c", subcore_axis_name="s")


@functools.partial(
    pl.kernel,
    mesh=_mesh,
    compiler_params=pltpu.CompilerParams(
        use_tc_tiling_on_sc=False,
        needs_layout_passes=False,
        disable_bounds_checks=True,
    ),
    out_type=jax.ShapeDtypeStruct((SEQ_LEN, 8, NB, 8, CHUNK), jnp.float32),
    scratch_types=[
        pltpu.VMEM((B_PER_W,), jnp.int32),
        pltpu.VMEM((NBUF * CHUNK, D_MODEL), jnp.float32),
        pltpu.VMEM((NBUF * 8, 8, CHUNK), jnp.float32),
        pltpu.SemaphoreType.DMA((NBUF,)),
        pltpu.SemaphoreType.DMA((NBUF,)),
    ],
)
def _emb_lookup(idx_hbm, table_hbm, out_hbm, idx_v, rows, tbs, gsem, wsem):
    wid = lax.axis_index("s") * NC + lax.axis_index("c")
    base = wid * B_PER_W

    pltpu.sync_copy(idx_hbm.at[pl.ds(base, B_PER_W)], idx_v)

    iota = lax.iota(jnp.int32, 16)

    def fire(j, b):
        pltpu.async_copy(
            table_hbm.at[idx_v.at[pl.ds(j * CHUNK, CHUNK)]],
            rows.at[pl.ds(b * CHUNK, CHUNK)],
            gsem.at[b],
        )

    def transpose(b):
        for d in range(D_MODEL):
            col = jnp.full((16,), d, jnp.int32)
            for g in range(8):
                v = plsc.load_gather(rows, [iota + (b * CHUNK + 16 * g), col])
                tbs[b * 8 + d // 8, d % 8, pl.ds(16 * g, 16)] = v

    def step(j, b):
        pltpu.make_async_copy(
            table_hbm.at[pl.ds(0, CHUNK)],
            rows.at[pl.ds(b * CHUNK, CHUNK)],
            gsem.at[b],
        ).wait()

        @pl.when(j >= NBUF)
        def _():
            pltpu.make_async_copy(
                table_hbm.at[pl.ds(0, CHUNK)],
                tbs.at[pl.ds(b * 8, 8)],
                wsem.at[b],
            ).wait()

        transpose(b)

        u = wid * U_PER_W + j
        s = u // NB
        c = lax.rem(u, NB)
        pltpu.async_copy(
            tbs.at[pl.ds(b * 8, 8)], out_hbm.at[s, :, c], wsem.at[b]
        )

        @pl.when(j < U_PER_W - NBUF)
        def _():
            fire(j + NBUF, b)

    for b in range(NBUF):
        fire(b, b)

    def group(g, carry):
        for b in range(NBUF):
            step(g * NBUF + b, b)
        return carry

    lax.fori_loop(0, U_PER_W // NBUF, group, 0, unroll=False)

    for b in range(NBUF):
        pltpu.make_async_copy(
            table_hbm.at[pl.ds(0, CHUNK)], tbs.at[pl.ds(b * 8, 8)], wsem.at[b]
        ).wait()


def kernel(char_tokens, table):
    idx = char_tokens.T.reshape(B).astype(jnp.int32)
    out5 = _emb_lookup(idx, table)
    return out5.transpose(2, 4, 0, 1, 3).reshape(BATCH, SEQ_LEN, D_MODEL)


# speedup vs baseline: 1.1524x; 1.1508x over previous
import functools

import jax
import jax.numpy as jnp
from jax import lax
from jax.experimental import pallas as pl
from jax.experimental.pallas import tpu as pltpu
from jax.experimental.pallas import tpu_sc as plsc

BATCH = 4096
SEQ_LEN = 200
D_MODEL = 64

NC = 2
NS = 16
NW = NC * NS

B = BATCH * SEQ_LEN
CHUNK = 128
NB = BATCH // CHUNK
UNITS = SEQ_LEN * NB
U_PER_W = UNITS // NW
B_PER_W = U_PER_W * CHUNK
NBUF = 2

_mesh = plsc.VectorSubcoreMesh(core_axis_name="c", subcore_axis_name="s")


@functools.partial(
    pl.kernel,
    mesh=_mesh,
    compiler_params=pltpu.CompilerParams(
        use_tc_tiling_on_sc=False,
        needs_layout_passes=False,
        disable_bounds_checks=True,
    ),
    out_type=jax.ShapeDtypeStruct((SEQ_LEN, 8, NB, 8, CHUNK), jnp.float32),
    scratch_types=[
        pltpu.VMEM((B_PER_W,), jnp.int32),
        pltpu.VMEM((NBUF * CHUNK, D_MODEL), jnp.float32),
        pltpu.VMEM((NBUF * 8, 8, CHUNK), jnp.float32),
        pltpu.SemaphoreType.DMA((NBUF,)),
        pltpu.SemaphoreType.DMA((NBUF,)),
    ],
)
def _emb_lookup(idx_hbm, table_hbm, out_hbm, idx_v, rows, tbs, gsem, wsem):
    wid = lax.axis_index("s") * NC + lax.axis_index("c")
    base = wid * B_PER_W

    pltpu.sync_copy(idx_hbm.at[pl.ds(base, B_PER_W)], idx_v)

    iota = lax.iota(jnp.int32, 16)

    def fire(j, b):
        pltpu.async_copy(
            table_hbm.at[idx_v.at[pl.ds(j * CHUNK, CHUNK)]],
            rows.at[pl.ds(b * CHUNK, CHUNK)],
            gsem.at[b],
        )

    def transpose(b):
        for d in range(D_MODEL):
            col = jnp.full((16,), d, jnp.int32)
            vs = [
                plsc.load_gather(rows, [iota + (b * CHUNK + 16 * g), col])
                for g in range(8)
            ]
            for g in range(8):
                tbs[b * 8 + d // 8, d % 8, pl.ds(16 * g, 16)] = vs[g]

    def step(j, b):
        pltpu.make_async_copy(
            table_hbm.at[pl.ds(0, CHUNK)],
            rows.at[pl.ds(b * CHUNK, CHUNK)],
            gsem.at[b],
        ).wait()

        @pl.when(j >= NBUF)
        def _():
            pltpu.make_async_copy(
                table_hbm.at[pl.ds(0, CHUNK)],
                tbs.at[pl.ds(b * 8, 8)],
                wsem.at[b],
            ).wait()

        transpose(b)

        u = wid * U_PER_W + j
        s = u // NB
        c = lax.rem(u, NB)
        pltpu.async_copy(
            tbs.at[pl.ds(b * 8, 8)], out_hbm.at[s, :, c], wsem.at[b]
        )

        @pl.when(j < U_PER_W - NBUF)
        def _():
            fire(j + NBUF, b)

    for b in range(NBUF):
        fire(b, b)

    def group(g, carry):
        for b in range(NBUF):
            step(g * NBUF + b, b)
        return carry

    lax.fori_loop(0, U_PER_W // NBUF, group, 0, unroll=False)

    for b in range(NBUF):
        pltpu.make_async_copy(
            table_hbm.at[pl.ds(0, CHUNK)], tbs.at[pl.ds(b * 8, 8)], wsem.at[b]
        ).wait()


def kernel(char_tokens, table):
    idx = char_tokens.T.reshape(B).astype(jnp.int32)
    out5 = _emb_lookup(idx, table)
    return out5.transpose(2, 4, 0, 1, 3).reshape(BATCH, SEQ_LEN, D_MODEL)


# speedup vs baseline: 2.0399x; 1.7701x over previous
import functools

import jax
import jax.numpy as jnp
from jax import lax
from jax.experimental import pallas as pl
from jax.experimental.pallas import tpu as pltpu
from jax.experimental.pallas import tpu_sc as plsc

BATCH = 4096
SEQ_LEN = 200
D_MODEL = 64

NC = 2
NS = 16
NW = NC * NS

B = BATCH * SEQ_LEN
CHUNK = 128
NB = BATCH // CHUNK
UNITS = SEQ_LEN * NB
U_PER_W = UNITS // NW
B_PER_W = U_PER_W * CHUNK
NBUF = 2

_mesh = plsc.VectorSubcoreMesh(core_axis_name="c", subcore_axis_name="s")


@functools.partial(
    pl.kernel,
    mesh=_mesh,
    compiler_params=pltpu.CompilerParams(
        use_tc_tiling_on_sc=False,
        needs_layout_passes=False,
        disable_bounds_checks=True,
    ),
    out_type=jax.ShapeDtypeStruct((SEQ_LEN, 8, NB, 8, CHUNK), jnp.float32),
    scratch_types=[
        pltpu.VMEM((B_PER_W,), jnp.int32),
        pltpu.VMEM((NBUF * CHUNK, D_MODEL), jnp.float32),
        pltpu.VMEM((NBUF * 8, 8, CHUNK), jnp.float32),
        pltpu.SemaphoreType.DMA((NBUF,)),
        pltpu.SemaphoreType.DMA((NBUF,)),
    ],
)
def _emb_lookup(idx_hbm, table_hbm, out_hbm, idx_v, rows, tbs, gsem, wsem):
    wid = lax.axis_index("s") * NC + lax.axis_index("c")
    base = wid * B_PER_W

    pltpu.sync_copy(idx_hbm.at[pl.ds(base, B_PER_W)], idx_v)

    iota = lax.iota(jnp.int32, 16)

    def fire(j, b):
        pltpu.async_copy(
            table_hbm.at[idx_v.at[pl.ds(j * CHUNK, CHUNK)]],
            rows.at[pl.ds(b * CHUNK, CHUNK)],
            gsem.at[b],
        )

    def transpose(b):
        for d in range(D_MODEL):
            col = jnp.full((16,), d, jnp.int32)
            vs = [
                plsc.load_gather(rows, [jnp.full((16,), b * CHUNK + 16 * g, jnp.int32), iota])
                for g in range(8)
            ]
            for g in range(8):
                tbs[b * 8 + d // 8, d % 8, pl.ds(16 * g, 16)] = vs[g]

    def step(j, b):
        pltpu.make_async_copy(
            table_hbm.at[pl.ds(0, CHUNK)],
            rows.at[pl.ds(b * CHUNK, CHUNK)],
            gsem.at[b],
        ).wait()

        @pl.when(j >= NBUF)
        def _():
            pltpu.make_async_copy(
                table_hbm.at[pl.ds(0, CHUNK)],
                tbs.at[pl.ds(b * 8, 8)],
                wsem.at[b],
            ).wait()

        transpose(b)

        u = wid * U_PER_W + j
        s = u // NB
        c = lax.rem(u, NB)
        pltpu.async_copy(
            tbs.at[pl.ds(b * 8, 8)], out_hbm.at[s, :, c], wsem.at[b]
        )

        @pl.when(j < U_PER_W - NBUF)
        def _():
            fire(j + NBUF, b)

    for b in range(NBUF):
        fire(b, b)

    def group(g, carry):
        for b in range(NBUF):
            step(g * NBUF + b, b)
        return carry

    lax.fori_loop(0, U_PER_W // NBUF, group, 0, unroll=False)

    for b in range(NBUF):
        pltpu.make_async_copy(
            table_hbm.at[pl.ds(0, CHUNK)], tbs.at[pl.ds(b * 8, 8)], wsem.at[b]
        ).wait()


def kernel(char_tokens, table):
    idx = char_tokens.T.reshape(B).astype(jnp.int32)
    out5 = _emb_lookup(idx, table)
    return out5.transpose(2, 4, 0, 1, 3).reshape(BATCH, SEQ_LEN, D_MODEL)


# speedup vs baseline: 2.2288x; 1.0926x over previous
import functools

import jax
import jax.numpy as jnp
from jax import lax
from jax.experimental import pallas as pl
from jax.experimental.pallas import tpu as pltpu
from jax.experimental.pallas import tpu_sc as plsc

BATCH = 4096
SEQ_LEN = 200
D_MODEL = 64

NC = 2
NS = 16
NW = NC * NS

B = BATCH * SEQ_LEN
CHUNK = 128
NB = BATCH // CHUNK
UNITS = SEQ_LEN * NB
U_PER_W = UNITS // NW
B_PER_W = U_PER_W * CHUNK
NBUF = 2

_mesh = plsc.VectorSubcoreMesh(core_axis_name="c", subcore_axis_name="s")


@functools.partial(
    pl.kernel,
    mesh=_mesh,
    compiler_params=pltpu.CompilerParams(
        use_tc_tiling_on_sc=False,
        needs_layout_passes=False,
        disable_bounds_checks=True,
    ),
    out_type=jax.ShapeDtypeStruct((SEQ_LEN, 8, NB, 8, CHUNK), jnp.float32),
    scratch_types=[
        pltpu.VMEM((B_PER_W,), jnp.int32),
        pltpu.VMEM((NBUF * CHUNK, D_MODEL), jnp.float32),
        pltpu.VMEM((NBUF * 8, 8, CHUNK), jnp.float32),
        pltpu.SemaphoreType.DMA((NBUF,)),
        pltpu.SemaphoreType.DMA((NBUF,)),
    ],
)
def _emb_lookup(idx_hbm, table_hbm, out_hbm, idx_v, rows, tbs, gsem, wsem):
    wid = lax.axis_index("s") * NC + lax.axis_index("c")
    base = wid * B_PER_W

    pltpu.sync_copy(idx_hbm.at[pl.ds(base, B_PER_W)], idx_v)

    iota = lax.iota(jnp.int32, 16)

    def fire(j, b):
        pltpu.async_copy(
            table_hbm.at[idx_v.at[pl.ds(j * CHUNK, CHUNK)]],
            rows.at[pl.ds(b * CHUNK, CHUNK)],
            gsem.at[b],
        )

    def transpose(b):
        def kblock(K, carry):
            for J in range(8):
                lane = iota + 16 * J
                rowv = lane + (b * CHUNK)
                for k0 in range(0, 16, 8):
                    grp = []
                    for k in range(k0, k0 + 8):
                        ro = (iota + k) % 16
                        colv = ro + 16 * K
                        v = plsc.load_gather(rows, [rowv, colv])
                        grp.append((ro // 8 + (2 * K + b * 8), ro % 8, v))
                    for row3, mid, v in grp:
                        plsc.store_scatter(tbs, [row3, mid, lane], v)
            return carry

        lax.fori_loop(0, 4, kblock, 0, unroll=False)

    def step(j, b):
        pltpu.make_async_copy(
            table_hbm.at[pl.ds(0, CHUNK)],
            rows.at[pl.ds(b * CHUNK, CHUNK)],
            gsem.at[b],
        ).wait()

        @pl.when(j >= NBUF)
        def _():
            pltpu.make_async_copy(
                table_hbm.at[pl.ds(0, CHUNK)],
                tbs.at[pl.ds(b * 8, 8)],
                wsem.at[b],
            ).wait()

        transpose(b)

        u = wid * U_PER_W + j
        s = u // NB
        c = lax.rem(u, NB)
        pltpu.async_copy(
            tbs.at[pl.ds(b * 8, 8)], out_hbm.at[s, :, c], wsem.at[b]
        )

        @pl.when(j < U_PER_W - NBUF)
        def _():
            fire(j + NBUF, b)

    for b in range(NBUF):
        fire(b, b)

    def group(g, carry):
        for b in range(NBUF):
            step(g * NBUF + b, b)
        return carry

    lax.fori_loop(0, U_PER_W // NBUF, group, 0, unroll=False)

    for b in range(NBUF):
        pltpu.make_async_copy(
            table_hbm.at[pl.ds(0, CHUNK)], tbs.at[pl.ds(b * 8, 8)], wsem.at[b]
        ).wait()


def kernel(char_tokens, table):
    idx = char_tokens.T.reshape(B).astype(jnp.int32)
    out5 = _emb_lookup(idx, table)
    return out5.transpose(2, 4, 0, 1, 3).reshape(BATCH, SEQ_LEN, D_MODEL)


# speedup vs baseline: 3.4973x; 1.5692x over previous
import functools

import jax
import jax.numpy as jnp
from jax import lax
from jax.experimental import pallas as pl
from jax.experimental.pallas import tpu as pltpu
from jax.experimental.pallas import tpu_sc as plsc

BATCH = 4096
SEQ_LEN = 200
D_MODEL = 64

NC = 2
NS = 16
NW = NC * NS

B = BATCH * SEQ_LEN
CHUNK = 128
NB = BATCH // CHUNK
UNITS = SEQ_LEN * NB
U_PER_W = UNITS // NW
B_PER_W = U_PER_W * CHUNK
NBUF = 2

VOCAB = 1000000
NSTRIPS = 7812
QS, RS = divmod(NSTRIPS, NW)
CB = 2

_mesh = plsc.VectorSubcoreMesh(core_axis_name="c", subcore_axis_name="s")


@functools.partial(
    pl.kernel,
    mesh=_mesh,
    compiler_params=pltpu.CompilerParams(
        use_tc_tiling_on_sc=True,
        needs_layout_passes=False,
        disable_bounds_checks=True,
    ),
    out_type=jax.ShapeDtypeStruct((VOCAB // 2, 128), jnp.float32),
    scratch_types=[
        pltpu.VMEM((CB * D_MODEL, 128), jnp.float32),
        pltpu.VMEM((CB * D_MODEL, 128), jnp.float32),
        pltpu.SemaphoreType.DMA((CB,)),
        pltpu.SemaphoreType.DMA((CB,)),
    ],
)
def _convert(tabT_hbm, tail_hbm, lin_hbm, buf, lin, rsem, wsem):
    wid = lax.axis_index("s") * NC + lax.axis_index("c")
    lo = wid * QS + jnp.minimum(wid, RS)
    cnt = QS + (wid < RS).astype(jnp.int32)
    iota = lax.iota(jnp.int32, 16)
    iota2 = iota + iota

    @pl.when(wid == NW - 1)
    def _():
        pltpu.sync_copy(tail_hbm, lin.at[pl.ds(0, 32)])
        pltpu.sync_copy(lin.at[pl.ds(0, 32)],
                        lin_hbm.at[pl.ds(VOCAB // 2 - 32, 32)])

    def fire(k, b):
        pltpu.async_copy(
            tabT_hbm.at[:, pl.ds((lo + k) * 128, 128)],
            buf.at[pl.ds(b * D_MODEL, D_MODEL)],
            rsem.at[b],
        )

    def transpose_strip(b):
        def kblock(K2, carry):
            d0 = 16 * K2
            for h in range(2):
                for J2 in range(4):
                    jj0 = 16 * J2
                    colv = iota2 + (2 * jj0 + h)
                    lrow = iota + (b * D_MODEL + jj0)
                    for k0 in range(0, 16, 8):
                        grp = []
                        for k in range(k0, k0 + 8):
                            rowv = (iota + k) % 16 + (d0 + b * D_MODEL)
                            v = plsc.load_gather(buf, [rowv, colv])
                            lcol = rowv + (h * 64 - b * D_MODEL)
                            grp.append((lcol, v))
                        for lcol, v in grp:
                            plsc.store_scatter(lin, [lrow, lcol], v)
            return carry

        lax.fori_loop(0, 4, kblock, 0, unroll=False)

    def step(k, b):
        pltpu.make_async_copy(
            lin_hbm.at[pl.ds(0, D_MODEL)],
            buf.at[pl.ds(b * D_MODEL, D_MODEL)],
            rsem.at[b],
        ).wait()

        @pl.when(k >= CB)
        def _():
            pltpu.make_async_copy(
                lin_hbm.at[pl.ds(0, D_MODEL)],
                lin.at[pl.ds(b * D_MODEL, D_MODEL)],
                wsem.at[b],
            ).wait()

        transpose_strip(b)

        pltpu.async_copy(
            lin.at[pl.ds(b * D_MODEL, D_MODEL)],
            lin_hbm.at[pl.ds(D_MODEL * (lo + k), D_MODEL)],
            wsem.at[b],
        )

        @pl.when(k < cnt - CB)
        def _():
            fire(k + CB, b)

    for b in range(CB):
        fire(b, b)

    def group(g, carry):
        for b in range(CB):
            step(g * CB + b, b)
        return carry

    lax.fori_loop(0, QS // CB, group, 0, unroll=False)

    @pl.when(wid < RS)
    def _():
        step(QS, 0)

    def drain(b, carry):
        pltpu.make_async_copy(
            lin_hbm.at[pl.ds(0, D_MODEL)],
            lin.at[pl.ds(b * D_MODEL, D_MODEL)],
            wsem.at[b],
        ).wait()
        return carry

    lax.fori_loop(0, CB, drain, 0, unroll=False)


@functools.partial(
    pl.kernel,
    mesh=_mesh,
    compiler_params=pltpu.CompilerParams(
        use_tc_tiling_on_sc=False,
        needs_layout_passes=False,
        disable_bounds_checks=True,
    ),
    out_type=jax.ShapeDtypeStruct((SEQ_LEN, 8, NB, 8, CHUNK), jnp.float32),
    scratch_types=[
        pltpu.VMEM((B_PER_W,), jnp.int32),
        pltpu.VMEM((NBUF * CHUNK, D_MODEL), jnp.float32),
        pltpu.VMEM((NBUF * 8, 8, CHUNK), jnp.float32),
        pltpu.SemaphoreType.DMA((NBUF,)),
        pltpu.SemaphoreType.DMA((NBUF,)),
    ],
)
def _emb_lookup(idx_hbm, table_hbm, out_hbm, idx_v, rows, tbs, gsem, wsem):
    wid = lax.axis_index("s") * NC + lax.axis_index("c")
    base = wid * B_PER_W

    pltpu.sync_copy(idx_hbm.at[pl.ds(base, B_PER_W)], idx_v)

    iota = lax.iota(jnp.int32, 16)

    def fire(j, b):
        pltpu.async_copy(
            table_hbm.at[idx_v.at[pl.ds(j * CHUNK, CHUNK)]],
            rows.at[pl.ds(b * CHUNK, CHUNK)],
            gsem.at[b],
        )

    def transpose(b):
        def kblock(K, carry):
            for J in range(8):
                lane = iota + 16 * J
                rowv = lane + (b * CHUNK)
                for k0 in range(0, 16, 8):
                    grp = []
                    for k in range(k0, k0 + 8):
                        ro = (iota + k) % 16
                        colv = ro + 16 * K
                        v = plsc.load_gather(rows, [rowv, colv])
                        grp.append((ro // 8 + (2 * K + b * 8), ro % 8, v))
                    for row3, mid, v in grp:
                        plsc.store_scatter(tbs, [row3, mid, lane], v)
            return carry

        lax.fori_loop(0, 4, kblock, 0, unroll=False)

    def step(j, b):
        pltpu.make_async_copy(
            table_hbm.at[pl.ds(0, CHUNK)],
            rows.at[pl.ds(b * CHUNK, CHUNK)],
            gsem.at[b],
        ).wait()

        @pl.when(j >= NBUF)
        def _():
            pltpu.make_async_copy(
                table_hbm.at[pl.ds(0, CHUNK)],
                tbs.at[pl.ds(b * 8, 8)],
                wsem.at[b],
            ).wait()

        transpose(b)

        u = wid * U_PER_W + j
        s = u // NB
        c = lax.rem(u, NB)
        pltpu.async_copy(
            tbs.at[pl.ds(b * 8, 8)], out_hbm.at[s, :, c], wsem.at[b]
        )

        @pl.when(j < U_PER_W - NBUF)
        def _():
            fire(j + NBUF, b)

    for b in range(NBUF):
        fire(b, b)

    def group(g, carry):
        for b in range(NBUF):
            step(g * NBUF + b, b)
        return carry

    lax.fori_loop(0, U_PER_W // NBUF, group, 0, unroll=False)

    for b in range(NBUF):
        pltpu.make_async_copy(
            table_hbm.at[pl.ds(0, CHUNK)], tbs.at[pl.ds(b * 8, 8)], wsem.at[b]
        ).wait()


def kernel(char_tokens, table):
    idx = char_tokens.T.reshape(B).astype(jnp.int32)
    tabT = table.T
    tail = lax.slice(table, (VOCAB - 64, 0), (VOCAB, D_MODEL)).reshape(32, 128)
    tab_lin = _convert(tabT, tail).reshape(VOCAB, D_MODEL)
    out5 = _emb_lookup(idx, tab_lin)
    return out5.transpose(2, 4, 0, 1, 3).reshape(BATCH, SEQ_LEN, D_MODEL)
